# baseline (speedup 1.0000x reference)
import functools

import jax
import jax.numpy as jnp
from jax import lax
from jax.experimental import pallas as pl
from jax.experimental.pallas import tpu as pltpu
from jax.experimental.pallas import tpu_sc as plsc

N_NODES = 100000
N_EDGES = 6400000
K = 4

_INFO = plsc.get_sparse_core_info()
NC, NS, L = _INFO.num_cores, _INFO.num_subcores, _INFO.num_lanes
NW = NC * NS

NPW = -(-N_NODES // NW)
CHW = -(-NPW // L)
NPW_PAD = CHW * L
OUT_W = NPW_PAD * K

ROW_LOG2 = 11
ROW_W = 1 << ROW_LOG2
EP_ROWS = N_EDGES // ROW_W

BLK = 8
PHASE_BLOCKS = ROW_W // BLK

RP_WIN = ((NPW_PAD + 1 + 7) // 8 + 1) * 8
RP_PAD_LEN = ((NW - 1) * NPW // 8) * 8 + RP_WIN

_NEG_INF = float("-inf")


def _sc_topk(rp_hbm, ed_hbm, outv_hbm, outi_hbm, rp_v, buf_v, idx_v, ov_v, oi_v, sem):
    wid = lax.axis_index("c") * NS + lax.axis_index("s")
    r0 = wid * NPW
    r0_al = pl.multiple_of(r0 & ~7, 8)
    off = r0 - r0_al
    pltpu.sync_copy(rp_hbm.at[pl.ds(r0_al, RP_WIN)], rp_v)

    lane = lax.iota(jnp.int32, L)
    lane2 = lane * 2
    neg = jnp.full((L,), _NEG_INF, jnp.float32)
    mone = jnp.full((L,), -1, jnp.int32)

    nl0 = lane * CHW
    lend = nl0 + CHW
    pos0 = plsc.load_gather(rp_v, [off + nl0])
    e0 = plsc.load_gather(rp_v, [off + nl0 + 1])
    rp_end = plsc.load_gather(rp_v, [off + lend])

    def step(e, pos, fa, v0, v1, v2, v3, i0, i1, i2, i3):
        active = pos < e
        row = fa >> ROW_LOG2
        col = fa & (ROW_W - 1)
        val = plsc.load_gather(buf_v, [row, col], mask=active)
        val = jnp.where(active, val, neg)
        b0 = val > v0
        b1 = val > v1
        b2 = val > v2
        b3 = val > v3
        nv0 = jnp.where(b0, val, v0)
        ni0 = jnp.where(b0, pos, i0)
        nv1 = jnp.where(b0, v0, jnp.where(b1, val, v1))
        ni1 = jnp.where(b0, i0, jnp.where(b1, pos, i1))
        nv2 = jnp.where(b1, v1, jnp.where(b2, val, v2))
        ni2 = jnp.where(b1, i1, jnp.where(b2, pos, i2))
        nv3 = jnp.where(b2, v2, jnp.where(b3, val, v3))
        ni3 = jnp.where(b2, i2, jnp.where(b3, pos, i3))
        inc = active.astype(jnp.int32)
        pos = pos + inc
        fa = fa + inc
        return pos, fa, nv0, nv1, nv2, nv3, ni0, ni1, ni2, ni3

    def phase_cond(c):
        nl = c[0]
        return jnp.max(lend - nl) > 0

    def phase_body(c):
        nl, pos, e, v0, v1, v2, v3, i0, i1, i2, i3 = c
        brow = pos >> ROW_LOG2
        plsc.store_scatter(idx_v, [lane2], brow)
        plsc.store_scatter(idx_v, [lane2 + 1],
                           jnp.minimum(brow + 1, EP_ROWS - 1))
        pltpu.async_copy(ed_hbm.at[idx_v], buf_v, sem).wait()
        fa0 = (lane << (ROW_LOG2 + 1)) + pos - (brow << ROW_LOG2)

        def block(b, c2):
            nl, pos, fa, e, v0, v1, v2, v3, i0, i1, i2, i3 = c2
            done = (pos >= e) & (nl < lend)
            addr = nl * K
            plsc.store_scatter(ov_v, [addr], v0, mask=done)
            plsc.store_scatter(ov_v, [addr + 1], v1, mask=done)
            plsc.store_scatter(ov_v, [addr + 2], v2, mask=done)
            plsc.store_scatter(ov_v, [addr + 3], v3, mask=done)
            plsc.store_scatter(oi_v, [addr], i0, mask=done)
            plsc.store_scatter(oi_v, [addr + 1], i1, mask=done)
            plsc.store_scatter(oi_v, [addr + 2], i2, mask=done)
            plsc.store_scatter(oi_v, [addr + 3], i3, mask=done)
            nl = nl + done.astype(jnp.int32)
            eg = plsc.load_gather(rp_v, [off + nl + 1])
            at_end = nl >= lend
            e = jnp.where(done, jnp.where(at_end, pos, eg), e)
            v0 = jnp.where(done, neg, v0)
            v1 = jnp.where(done, neg, v1)
            v2 = jnp.where(done, neg, v2)
            v3 = jnp.where(done, neg, v3)
            i0 = jnp.where(done, mone, i0)
            i1 = jnp.where(done, mone, i1)
            i2 = jnp.where(done, mone, i2)
            i3 = jnp.where(done, mone, i3)
            st = (pos, fa, v0, v1, v2, v3, i0, i1, i2, i3)
            for _ in range(BLK):
                st = step(e, *st)
            pos, fa, v0, v1, v2, v3, i0, i1, i2, i3 = st
            return (nl, pos, fa, e, v0, v1, v2, v3, i0, i1, i2, i3)

        nl, pos, e, v0, v1, v2, v3, i0, i1, i2, i3 = c
        rem = jnp.max((rp_end - pos) + (lend - nl))
        trip = jnp.minimum(PHASE_BLOCKS, (rem + BLK - 1) // BLK)
        res = lax.fori_loop(
            0, trip, block,
            (nl, pos, fa0, e, v0, v1, v2, v3, i0, i1, i2, i3))
        nl, pos, _, e, v0, v1, v2, v3, i0, i1, i2, i3 = res
        return (nl, pos, e, v0, v1, v2, v3, i0, i1, i2, i3)

    carry = (nl0, pos0, e0, neg, neg, neg, neg, mone, mone, mone, mone)
    lax.while_loop(phase_cond, phase_body, carry)

    pltpu.sync_copy(ov_v, outv_hbm.at[wid])
    pltpu.sync_copy(oi_v, outi_hbm.at[wid])


_sc_call = functools.partial(
    pl.kernel,
    mesh=plsc.VectorSubcoreMesh(core_axis_name="c", subcore_axis_name="s"),
    compiler_params=pltpu.CompilerParams(
        needs_layout_passes=False, use_tc_tiling_on_sc=False),
    out_type=[
        jax.ShapeDtypeStruct((NW, OUT_W), jnp.float32),
        jax.ShapeDtypeStruct((NW, OUT_W), jnp.int32),
    ],
    scratch_types=[
        pltpu.VMEM((RP_WIN,), jnp.int32),
        pltpu.VMEM((2 * L, ROW_W), jnp.float32),
        pltpu.VMEM((2 * L,), jnp.int32),
        pltpu.VMEM((OUT_W,), jnp.float32),
        pltpu.VMEM((OUT_W,), jnp.int32),
        pltpu.SemaphoreType.DMA,
    ],
)(_sc_topk)


@jax.jit
def kernel(row_ptr, edge_scores):
    rp32 = row_ptr.astype(jnp.int32)
    rp = jnp.concatenate(
        [rp32,
         jnp.broadcast_to(rp32[-1], (RP_PAD_LEN - (N_NODES + 1),))]
    )
    ed = edge_scores.reshape(EP_ROWS, ROW_W)
    outv, outi = _sc_call(rp, ed)
    vals = outv.reshape(NW, NPW_PAD, K)[:, :NPW, :].reshape(N_NODES, K)
    idxs = outi.reshape(NW, NPW_PAD, K)[:, :NPW, :].reshape(N_NODES, K)
    return vals, idxs.astype(jnp.int64)


# speedup vs baseline: 1.0267x; 1.0267x over previous
import functools

import jax
import jax.numpy as jnp
from jax import lax
from jax.experimental import pallas as pl
from jax.experimental.pallas import tpu as pltpu
from jax.experimental.pallas import tpu_sc as plsc

N_NODES = 100000
N_EDGES = 6400000
K = 4

_INFO = plsc.get_sparse_core_info()
NC, NS, L = _INFO.num_cores, _INFO.num_subcores, _INFO.num_lanes
NW = NC * NS

NPW = -(-N_NODES // NW)
CHW = -(-NPW // L)
NPW_PAD = CHW * L
OUT_W = NPW_PAD * K

ROW_LOG2 = 11
ROW_W = 1 << ROW_LOG2
EP_ROWS = N_EDGES // ROW_W

BLK = 8
PHASE_BLOCKS = ROW_W // BLK

RP_WIN = ((NPW_PAD + 1 + 7) // 8 + 1) * 8
RP_PAD_LEN = ((NW - 1) * NPW // 8) * 8 + RP_WIN

_NEG_INF = float("-inf")


def _sc_topk(rp_hbm, ed_hbm, outv_hbm, outi_hbm, rp_v, buf_v, idx_v, ov_v, oi_v, sem):
    wid = lax.axis_index("c") * NS + lax.axis_index("s")
    r0 = wid * NPW
    r0_al = pl.multiple_of(r0 & ~7, 8)
    off = r0 - r0_al
    pltpu.sync_copy(rp_hbm.at[pl.ds(r0_al, RP_WIN)], rp_v)

    lane = lax.iota(jnp.int32, L)
    lane2 = lane * 2
    neg = jnp.full((L,), _NEG_INF, jnp.float32)
    mone = jnp.full((L,), -1, jnp.int32)

    nl0 = lane * CHW
    lend = nl0 + CHW
    pos0 = plsc.load_gather(rp_v, [off + nl0])
    e0 = plsc.load_gather(rp_v, [off + nl0 + 1])

    def step(e, pos, fa, v0, v1, v2, v3, i0, i1, i2, i3):
        active = pos < e
        row = fa >> ROW_LOG2
        col = fa & (ROW_W - 1)
        val = plsc.load_gather(buf_v, [row, col], mask=active)
        val = jnp.where(active, val, neg)
        b0 = val > v0
        b1 = val > v1
        b2 = val > v2
        b3 = val > v3
        nv0 = jnp.where(b0, val, v0)
        ni0 = jnp.where(b0, pos, i0)
        nv1 = jnp.where(b0, v0, jnp.where(b1, val, v1))
        ni1 = jnp.where(b0, i0, jnp.where(b1, pos, i1))
        nv2 = jnp.where(b1, v1, jnp.where(b2, val, v2))
        ni2 = jnp.where(b1, i1, jnp.where(b2, pos, i2))
        nv3 = jnp.where(b2, v2, jnp.where(b3, val, v3))
        ni3 = jnp.where(b2, i2, jnp.where(b3, pos, i3))
        inc = active.astype(jnp.int32)
        pos = pos + inc
        fa = fa + inc
        return pos, fa, nv0, nv1, nv2, nv3, ni0, ni1, ni2, ni3

    def phase_cond(c):
        nl = c[0]
        return jnp.max(lend - nl) > 0

    def phase_body(c):
        nl, pos, e, v0, v1, v2, v3, i0, i1, i2, i3 = c
        brow = pos >> ROW_LOG2
        plsc.store_scatter(idx_v, [lane2], brow)
        plsc.store_scatter(idx_v, [lane2 + 1],
                           jnp.minimum(brow + 1, EP_ROWS - 1))
        pltpu.async_copy(ed_hbm.at[idx_v], buf_v, sem).wait()
        fa0 = (lane << (ROW_LOG2 + 1)) + pos - (brow << ROW_LOG2)

        def block(b, c2):
            nl, pos, fa, e, v0, v1, v2, v3, i0, i1, i2, i3 = c2
            done = (pos >= e) & (nl < lend)
            addr = nl * K
            plsc.store_scatter(ov_v, [addr], v0, mask=done)
            plsc.store_scatter(ov_v, [addr + 1], v1, mask=done)
            plsc.store_scatter(ov_v, [addr + 2], v2, mask=done)
            plsc.store_scatter(ov_v, [addr + 3], v3, mask=done)
            plsc.store_scatter(oi_v, [addr], i0, mask=done)
            plsc.store_scatter(oi_v, [addr + 1], i1, mask=done)
            plsc.store_scatter(oi_v, [addr + 2], i2, mask=done)
            plsc.store_scatter(oi_v, [addr + 3], i3, mask=done)
            nl = nl + done.astype(jnp.int32)
            eg = plsc.load_gather(rp_v, [off + nl + 1])
            at_end = nl >= lend
            e = jnp.where(done, jnp.where(at_end, pos, eg), e)
            v0 = jnp.where(done, neg, v0)
            v1 = jnp.where(done, neg, v1)
            v2 = jnp.where(done, neg, v2)
            v3 = jnp.where(done, neg, v3)
            i0 = jnp.where(done, mone, i0)
            i1 = jnp.where(done, mone, i1)
            i2 = jnp.where(done, mone, i2)
            i3 = jnp.where(done, mone, i3)
            st = (pos, fa, v0, v1, v2, v3, i0, i1, i2, i3)
            for _ in range(BLK):
                st = step(e, *st)
            pos, fa, v0, v1, v2, v3, i0, i1, i2, i3 = st
            return (nl, pos, fa, e, v0, v1, v2, v3, i0, i1, i2, i3)

        nl, pos, e, v0, v1, v2, v3, i0, i1, i2, i3 = c
        res = lax.fori_loop(
            0, PHASE_BLOCKS, block,
            (nl, pos, fa0, e, v0, v1, v2, v3, i0, i1, i2, i3))
        nl, pos, _, e, v0, v1, v2, v3, i0, i1, i2, i3 = res
        return (nl, pos, e, v0, v1, v2, v3, i0, i1, i2, i3)

    carry = (nl0, pos0, e0, neg, neg, neg, neg, mone, mone, mone, mone)
    lax.while_loop(phase_cond, phase_body, carry)

    pltpu.sync_copy(ov_v, outv_hbm.at[wid])
    pltpu.sync_copy(oi_v, outi_hbm.at[wid])


_sc_call = functools.partial(
    pl.kernel,
    mesh=plsc.VectorSubcoreMesh(core_axis_name="c", subcore_axis_name="s"),
    compiler_params=pltpu.CompilerParams(
        needs_layout_passes=False, use_tc_tiling_on_sc=False),
    out_type=[
        jax.ShapeDtypeStruct((NW, OUT_W), jnp.float32),
        jax.ShapeDtypeStruct((NW, OUT_W), jnp.int32),
    ],
    scratch_types=[
        pltpu.VMEM((RP_WIN,), jnp.int32),
        pltpu.VMEM((2 * L, ROW_W), jnp.float32),
        pltpu.VMEM((2 * L,), jnp.int32),
        pltpu.VMEM((OUT_W,), jnp.float32),
        pltpu.VMEM((OUT_W,), jnp.int32),
        pltpu.SemaphoreType.DMA,
    ],
)(_sc_topk)


@jax.jit
def kernel(row_ptr, edge_scores):
    rp32 = row_ptr.astype(jnp.int32)
    rp = jnp.concatenate(
        [rp32,
         jnp.broadcast_to(rp32[-1], (RP_PAD_LEN - (N_NODES + 1),))]
    )
    ed = edge_scores.reshape(EP_ROWS, ROW_W)
    outv, outi = _sc_call(rp, ed)
    vals = outv.reshape(NW, NPW_PAD, K)[:, :NPW, :].reshape(N_NODES, K)
    idxs = outi.reshape(NW, NPW_PAD, K)[:, :NPW, :].reshape(N_NODES, K)
    return vals, idxs.astype(jnp.int64)


# speedup vs baseline: 1.0903x; 1.0620x over previous
import functools

import jax
import jax.numpy as jnp
from jax import lax
from jax.experimental import pallas as pl
from jax.experimental.pallas import tpu as pltpu
from jax.experimental.pallas import tpu_sc as plsc

N_NODES = 100000
N_EDGES = 6400000
K = 4

_INFO = plsc.get_sparse_core_info()
NC, NS, L = _INFO.num_cores, _INFO.num_subcores, _INFO.num_lanes
NW = NC * NS

NPW = -(-N_NODES // NW)
CHW = -(-NPW // L)
NPW_PAD = CHW * L
OUT_W = NPW_PAD * K

ROW_LOG2 = 11
ROW_W = 1 << ROW_LOG2
EP_ROWS = N_EDGES // ROW_W

BLK = 12
PHASE_BLOCKS = ROW_W // BLK

RP_WIN = ((NPW_PAD + 1 + 7) // 8 + 1) * 8
RP_PAD_LEN = ((NW - 1) * NPW // 8) * 8 + RP_WIN

_NEG_INF = float("-inf")


def _sc_topk(rp_hbm, ed_hbm, outv_hbm, outi_hbm, rp_v, buf_v, idx_v, ov_v, oi_v, sem):
    wid = lax.axis_index("c") * NS + lax.axis_index("s")
    r0 = wid * NPW
    r0_al = pl.multiple_of(r0 & ~7, 8)
    off = r0 - r0_al
    pltpu.sync_copy(rp_hbm.at[pl.ds(r0_al, RP_WIN)], rp_v)

    lane = lax.iota(jnp.int32, L)
    lane2 = lane * 2
    neg = jnp.full((L,), _NEG_INF, jnp.float32)
    mone = jnp.full((L,), -1, jnp.int32)

    nl0 = lane * CHW
    lend = nl0 + CHW
    pos0 = plsc.load_gather(rp_v, [off + nl0])
    e0 = plsc.load_gather(rp_v, [off + nl0 + 1])

    def step(e, pos, fa, v0, v1, v2, v3, i0, i1, i2, i3):
        active = pos < e
        row = fa >> ROW_LOG2
        col = fa & (ROW_W - 1)
        val = plsc.load_gather(buf_v, [row, col], mask=active)
        val = jnp.where(active, val, neg)
        b0 = val > v0
        b1 = val > v1
        b2 = val > v2
        b3 = val > v3
        nv0 = jnp.where(b0, val, v0)
        ni0 = jnp.where(b0, pos, i0)
        nv1 = jnp.where(b0, v0, jnp.where(b1, val, v1))
        ni1 = jnp.where(b0, i0, jnp.where(b1, pos, i1))
        nv2 = jnp.where(b1, v1, jnp.where(b2, val, v2))
        ni2 = jnp.where(b1, i1, jnp.where(b2, pos, i2))
        nv3 = jnp.where(b2, v2, jnp.where(b3, val, v3))
        ni3 = jnp.where(b2, i2, jnp.where(b3, pos, i3))
        inc = active.astype(jnp.int32)
        pos = pos + inc
        fa = fa + inc
        return pos, fa, nv0, nv1, nv2, nv3, ni0, ni1, ni2, ni3

    def phase_cond(c):
        nl = c[0]
        return jnp.max(lend - nl) > 0

    def phase_body(c):
        nl, pos, e, v0, v1, v2, v3, i0, i1, i2, i3 = c
        brow = pos >> ROW_LOG2
        plsc.store_scatter(idx_v, [lane2], brow)
        plsc.store_scatter(idx_v, [lane2 + 1],
                           jnp.minimum(brow + 1, EP_ROWS - 1))
        pltpu.async_copy(ed_hbm.at[idx_v], buf_v, sem).wait()
        fa0 = (lane << (ROW_LOG2 + 1)) + pos - (brow << ROW_LOG2)

        def block(b, c2):
            nl, pos, fa, e, v0, v1, v2, v3, i0, i1, i2, i3 = c2
            done = (pos >= e) & (nl < lend)
            addr = nl * K
            plsc.store_scatter(ov_v, [addr], v0, mask=done)
            plsc.store_scatter(ov_v, [addr + 1], v1, mask=done)
            plsc.store_scatter(ov_v, [addr + 2], v2, mask=done)
            plsc.store_scatter(ov_v, [addr + 3], v3, mask=done)
            plsc.store_scatter(oi_v, [addr], i0, mask=done)
            plsc.store_scatter(oi_v, [addr + 1], i1, mask=done)
            plsc.store_scatter(oi_v, [addr + 2], i2, mask=done)
            plsc.store_scatter(oi_v, [addr + 3], i3, mask=done)
            nl = nl + done.astype(jnp.int32)
            eg = plsc.load_gather(rp_v, [off + nl + 1])
            at_end = nl >= lend
            e = jnp.where(done, jnp.where(at_end, pos, eg), e)
            v0 = jnp.where(done, neg, v0)
            v1 = jnp.where(done, neg, v1)
            v2 = jnp.where(done, neg, v2)
            v3 = jnp.where(done, neg, v3)
            i0 = jnp.where(done, mone, i0)
            i1 = jnp.where(done, mone, i1)
            i2 = jnp.where(done, mone, i2)
            i3 = jnp.where(done, mone, i3)
            st = (pos, fa, v0, v1, v2, v3, i0, i1, i2, i3)
            for _ in range(BLK):
                st = step(e, *st)
            pos, fa, v0, v1, v2, v3, i0, i1, i2, i3 = st
            return (nl, pos, fa, e, v0, v1, v2, v3, i0, i1, i2, i3)

        nl, pos, e, v0, v1, v2, v3, i0, i1, i2, i3 = c
        res = lax.fori_loop(
            0, PHASE_BLOCKS, block,
            (nl, pos, fa0, e, v0, v1, v2, v3, i0, i1, i2, i3))
        nl, pos, _, e, v0, v1, v2, v3, i0, i1, i2, i3 = res
        return (nl, pos, e, v0, v1, v2, v3, i0, i1, i2, i3)

    carry = (nl0, pos0, e0, neg, neg, neg, neg, mone, mone, mone, mone)
    lax.while_loop(phase_cond, phase_body, carry)

    pltpu.sync_copy(ov_v, outv_hbm.at[wid])
    pltpu.sync_copy(oi_v, outi_hbm.at[wid])


_sc_call = functools.partial(
    pl.kernel,
    mesh=plsc.VectorSubcoreMesh(core_axis_name="c", subcore_axis_name="s"),
    compiler_params=pltpu.CompilerParams(
        needs_layout_passes=False, use_tc_tiling_on_sc=False),
    out_type=[
        jax.ShapeDtypeStruct((NW, OUT_W), jnp.float32),
        jax.ShapeDtypeStruct((NW, OUT_W), jnp.int32),
    ],
    scratch_types=[
        pltpu.VMEM((RP_WIN,), jnp.int32),
        pltpu.VMEM((2 * L, ROW_W), jnp.float32),
        pltpu.VMEM((2 * L,), jnp.int32),
        pltpu.VMEM((OUT_W,), jnp.float32),
        pltpu.VMEM((OUT_W,), jnp.int32),
        pltpu.SemaphoreType.DMA,
    ],
)(_sc_topk)


@jax.jit
def kernel(row_ptr, edge_scores):
    rp32 = row_ptr.astype(jnp.int32)
    rp = jnp.concatenate(
        [rp32,
         jnp.broadcast_to(rp32[-1], (RP_PAD_LEN - (N_NODES + 1),))]
    )
    ed = edge_scores.reshape(EP_ROWS, ROW_W)
    outv, outi = _sc_call(rp, ed)
    vals = outv.reshape(NW, NPW_PAD, K)[:, :NPW, :].reshape(N_NODES, K)
    idxs = outi.reshape(NW, NPW_PAD, K)[:, :NPW, :].reshape(N_NODES, K)
    return vals, idxs.astype(jnp.int64)
